# baseline (speedup 1.0000x reference)
import functools

import jax
import jax.numpy as jnp
import numpy as np
from jax import lax
from jax.experimental import pallas as pl
from jax.experimental.pallas import tpu as pltpu
from jax.experimental.pallas import tpu_sc as plsc

B, H, W, C = 4, 224, 224, 192
HW = H * W
NPIX = B * HW
LANES = 16
CH = 64
GROUPS = CH // LANES
NTILES = 32
PTS_PER_TILE = NPIX // NTILES
NCHUNKS = PTS_PER_TILE // CH
CVECS = C // LANES
PROW = 1024


def _corner_slice(corner, cv):
    flat = corner * C + cv * LANES
    return flat // 128, flat % 128


def _tec_body(patch, xs_hbm, ys_hbm, out,
              xsv, ysv, idxv, wav, wbv, wcv, wdv, bufp, outb, gsem):
    c_id = lax.axis_index("c")
    s_id = lax.axis_index("s")
    wid = s_id * 2 + c_id
    base0 = wid * PTS_PER_TILE
    batch = wid // (NTILES // B)
    bb = batch * HW

    def chunk_body(t, _):
        start = base0 + t * CH
        pltpu.sync_copy(xs_hbm.at[pl.ds(start, CH)], xsv)
        pltpu.sync_copy(ys_hbm.at[pl.ds(start, CH)], ysv)
        for g in range(GROUPS):
            sl = pl.ds(g * LANES, LANES)
            xs = xsv[sl]
            ys = ysv[sl]
            x0 = xs.astype(jnp.int32)
            y0 = ys.astype(jnp.int32)
            x0c = jnp.clip(x0, 0, W - 1)
            x1c = jnp.clip(x0 + 1, 0, W - 1)
            y0c = jnp.clip(y0, 0, H - 1)
            y1c = jnp.clip(y0 + 1, 0, H - 1)
            x0f = x0c.astype(jnp.float32)
            x1f = x1c.astype(jnp.float32)
            y0f = y0c.astype(jnp.float32)
            y1f = y1c.astype(jnp.float32)
            zero = jnp.zeros((LANES,), jnp.float32)
            eqx = x0c == x1c
            eqy = y0c == y1c
            wxl = jnp.where(eqx, zero, x1f - xs)
            wxr = jnp.where(eqx, zero, xs - x0f)
            wyt = jnp.where(eqy, zero, y1f - ys)
            wyb = jnp.where(eqy, zero, ys - y0f)
            wav[sl] = wxl * wyt
            wbv[sl] = wxl * wyb
            wcv[sl] = wxr * wyt
            wdv[sl] = wxr * wyb
            idxv[sl] = bb + y0c * W + x0c
        pltpu.async_copy(patch.at[idxv], bufp, gsem).wait()

        @plsc.parallel_loop(0, CH, step=1, unroll=4)
        def pt_body(p):
            pidx = jnp.full((LANES,), p, jnp.int32)
            wa = plsc.load_gather(wav, [pidx])
            wb = plsc.load_gather(wbv, [pidx])
            wc = plsc.load_gather(wcv, [pidx])
            wd = plsc.load_gather(wdv, [pidx])
            for cv in range(CVECS):
                ra, ca = _corner_slice(0, cv)
                rc, cc = _corner_slice(1, cv)
                rb, cb = _corner_slice(2, cv)
                rd, cd = _corner_slice(3, cv)
                acc = ((wa * bufp[p, ra, pl.ds(ca, LANES)]
                        + wb * bufp[p, rb, pl.ds(cb, LANES)])
                       + wc * bufp[p, rc, pl.ds(cc, LANES)]) \
                    + wd * bufp[p, rd, pl.ds(cd, LANES)]
                outb[p, pl.ds(cv * LANES, LANES)] = acc

        pltpu.sync_copy(outb, out.at[pl.ds(start, CH)])
        return 0

    lax.fori_loop(0, NCHUNKS, chunk_body, 0)


@jax.jit
def _sc_interp(patch, xs_flat, ys_flat):
    mesh = plsc.VectorSubcoreMesh(core_axis_name="c", subcore_axis_name="s")
    fn = pl.kernel(
        _tec_body,
        mesh=mesh,
        compiler_params=pltpu.CompilerParams(
            needs_layout_passes=False, use_tc_tiling_on_sc=True),
        out_type=jax.ShapeDtypeStruct((NPIX, C), jnp.float32),
        scratch_types=[
            pltpu.VMEM((CH,), jnp.float32),
            pltpu.VMEM((CH,), jnp.float32),
            pltpu.VMEM((CH,), jnp.int32),
            pltpu.VMEM((CH,), jnp.float32),
            pltpu.VMEM((CH,), jnp.float32),
            pltpu.VMEM((CH,), jnp.float32),
            pltpu.VMEM((CH,), jnp.float32),
            pltpu.VMEM((CH, 8, 128), jnp.float32),
            pltpu.VMEM((CH, C), jnp.float32),
            pltpu.SemaphoreType.DMA,
        ],
    )
    return fn(patch, xs_flat, ys_flat)


def kernel(X, transformation):
    x_linspace = jnp.linspace(-1.0, 1.0, W)
    y_linspace = jnp.linspace(-1.0, 1.0, H)
    x_coordinates, y_coordinates = jnp.meshgrid(x_linspace, y_linspace)
    x_coordinates = x_coordinates.reshape(-1)
    y_coordinates = y_coordinates.reshape(-1)
    ones = jnp.ones_like(x_coordinates)
    grid = jnp.concatenate([x_coordinates, y_coordinates, ones], axis=0)
    grids = jnp.tile(grid.reshape(-1), (B,)).reshape(B, 3, HW)
    transformations = transformation.reshape(B, 2, 3)
    sampled_grids = jnp.einsum('bij,bjk->bik', transformations, grids)
    x = sampled_grids[:, 0:1, :].reshape(-1).astype(jnp.float32)
    y = sampled_grids[:, 1:2, :].reshape(-1).astype(jnp.float32)
    x = 0.5 * (x + 1.0) * jnp.float32(H)
    y = 0.5 * (y + 1.0) * jnp.float32(W)

    imgf = X.reshape(NPIX, C)
    zrow = jnp.zeros((225, C), jnp.float32)
    sh1 = jnp.concatenate([imgf[1:], zrow[:1]], axis=0)
    sh224 = jnp.concatenate([imgf[W:], zrow[:W]], axis=0)
    sh225 = jnp.concatenate([imgf[W + 1:], zrow], axis=0)
    pad = jnp.zeros((NPIX, PROW - 4 * C), jnp.float32)
    patch = jnp.concatenate([imgf, sh1, sh224, sh225, pad], axis=1)
    patch = patch.reshape(NPIX, 8, 128)
    one = transformation.reshape(-1)[0] * 0.0 + 1.0
    patch = patch * one

    out = _sc_interp(patch, x, y)
    return out.reshape(B, H, W, C)


# speedup vs baseline: 1.1521x; 1.1521x over previous
import functools

import jax
import jax.numpy as jnp
import numpy as np
from jax import lax
from jax.experimental import pallas as pl
from jax.experimental.pallas import tpu as pltpu
from jax.experimental.pallas import tpu_sc as plsc

B, H, W, C = 4, 224, 224, 192
HW = H * W
NPIX = B * HW
LANES = 16
CH = 64
GROUPS = CH // LANES
NTILES = 32
PTS_PER_TILE = NPIX // NTILES
NCHUNKS = PTS_PER_TILE // CH
CVECS = C // LANES


def _tec_body(img, xs_hbm, ys_hbm, out,
              xsva, ysva, idxa_a, idxb_a, idxc_a, idxd_a,
              wav_a, wbv_a, wcv_a, wdv_a,
              bufa_a, bufb_a, bufc_a, bufd_a, outb_a,
              xsvb, ysvb, idxa_b, idxb_b, idxc_b, idxd_b,
              wav_b, wbv_b, wcv_b, wdv_b,
              bufa_b, bufb_b, bufc_b, bufd_b, outb_b,
              gsema, gsemb):
    c_id = lax.axis_index("c")
    s_id = lax.axis_index("s")
    wid = s_id * 2 + c_id
    base0 = wid * PTS_PER_TILE
    batch = wid // (NTILES // B)
    bb = batch * HW

    def emit_idx(t, xsv, ysv, ia, ib, ic, idd, wa_r, wb_r, wc_r, wd_r):
        start = base0 + t * CH
        pltpu.sync_copy(xs_hbm.at[pl.ds(start, CH)], xsv)
        pltpu.sync_copy(ys_hbm.at[pl.ds(start, CH)], ysv)
        for g in range(GROUPS):
            sl = pl.ds(g * LANES, LANES)
            xs = xsv[sl]
            ys = ysv[sl]
            x0 = xs.astype(jnp.int32)
            y0 = ys.astype(jnp.int32)
            x0c = jnp.clip(x0, 0, H - 1)
            x1c = jnp.clip(x0 + 1, 0, H - 1)
            y0c = jnp.clip(y0, 0, W - 1)
            y1c = jnp.clip(y0 + 1, 0, W - 1)
            x0f = x0c.astype(jnp.float32)
            x1f = x1c.astype(jnp.float32)
            y0f = y0c.astype(jnp.float32)
            y1f = y1c.astype(jnp.float32)
            wxl = x1f - xs
            wxr = xs - x0f
            wyt = y1f - ys
            wyb = ys - y0f
            wa_r[sl] = wxl * wyt
            wb_r[sl] = wxl * wyb
            wc_r[sl] = wxr * wyt
            wd_r[sl] = wxr * wyb
            ia[sl] = bb + y0c * W + x0c
            ib[sl] = bb + y1c * W + x0c
            ic[sl] = bb + y0c * W + x1c
            idd[sl] = bb + y1c * W + x1c

    def fire(ia, ib, ic, idd, ba, bbuf, bc, bd, sem):
        pltpu.async_copy(img.at[ia], ba, sem)
        pltpu.async_copy(img.at[ib], bbuf, sem)
        pltpu.async_copy(img.at[ic], bc, sem)
        pltpu.async_copy(img.at[idd], bd, sem)

    def drain(ia, ib, ic, idd, ba, bbuf, bc, bd, sem):
        pltpu.make_async_copy(img.at[ia], ba, sem).wait()
        pltpu.make_async_copy(img.at[ib], bbuf, sem).wait()
        pltpu.make_async_copy(img.at[ic], bc, sem).wait()
        pltpu.make_async_copy(img.at[idd], bd, sem).wait()

    def combine(t, wa_r, wb_r, wc_r, wd_r, ba, bbuf, bc, bd, outb):
        @plsc.parallel_loop(0, CH, step=1, unroll=4)
        def pt_body(p):
            pidx = jnp.full((LANES,), p, jnp.int32)
            wa = plsc.load_gather(wa_r, [pidx])
            wb = plsc.load_gather(wb_r, [pidx])
            wc = plsc.load_gather(wc_r, [pidx])
            wd = plsc.load_gather(wd_r, [pidx])
            for cv in range(CVECS):
                sl = pl.ds(cv * LANES, LANES)
                acc = ((wa * ba[p, sl] + wb * bbuf[p, sl])
                       + wc * bc[p, sl]) + wd * bd[p, sl]
                outb[p, sl] = acc

        pltpu.sync_copy(outb, out.at[pl.ds(base0 + t * CH, CH)])

    seta_idx = (idxa_a, idxb_a, idxc_a, idxd_a)
    seta_buf = (bufa_a, bufb_a, bufc_a, bufd_a)
    seta_w = (wav_a, wbv_a, wcv_a, wdv_a)
    setb_idx = (idxa_b, idxb_b, idxc_b, idxd_b)
    setb_buf = (bufa_b, bufb_b, bufc_b, bufd_b)
    setb_w = (wav_b, wbv_b, wcv_b, wdv_b)

    emit_idx(0, xsva, ysva, *seta_idx, *seta_w)
    fire(*seta_idx, *seta_buf, gsema)

    def pair_body(k, _):
        ta = 2 * k
        tb = ta + 1
        emit_idx(tb, xsvb, ysvb, *setb_idx, *setb_w)
        fire(*setb_idx, *setb_buf, gsemb)
        drain(*seta_idx, *seta_buf, gsema)
        combine(ta, *seta_w, *seta_buf, outb_a)

        @pl.when(k < NCHUNKS // 2 - 1)
        def _():
            emit_idx(ta + 2, xsva, ysva, *seta_idx, *seta_w)
            fire(*seta_idx, *seta_buf, gsema)

        drain(*setb_idx, *setb_buf, gsemb)
        combine(tb, *setb_w, *setb_buf, outb_b)
        return 0

    lax.fori_loop(0, NCHUNKS // 2, pair_body, 0)


@jax.jit
def _sc_interp(img, xs_flat, ys_flat):
    mesh = plsc.VectorSubcoreMesh(core_axis_name="c", subcore_axis_name="s")

    def dbuf():
        return [
            pltpu.VMEM((CH,), jnp.float32),
            pltpu.VMEM((CH,), jnp.float32),
            pltpu.VMEM((CH,), jnp.int32),
            pltpu.VMEM((CH,), jnp.int32),
            pltpu.VMEM((CH,), jnp.int32),
            pltpu.VMEM((CH,), jnp.int32),
            pltpu.VMEM((CH,), jnp.float32),
            pltpu.VMEM((CH,), jnp.float32),
            pltpu.VMEM((CH,), jnp.float32),
            pltpu.VMEM((CH,), jnp.float32),
            pltpu.VMEM((CH, C), jnp.float32),
            pltpu.VMEM((CH, C), jnp.float32),
            pltpu.VMEM((CH, C), jnp.float32),
            pltpu.VMEM((CH, C), jnp.float32),
            pltpu.VMEM((CH, C), jnp.float32),
        ]

    fn = pl.kernel(
        _tec_body,
        mesh=mesh,
        compiler_params=pltpu.CompilerParams(
            needs_layout_passes=False, use_tc_tiling_on_sc=False),
        out_type=jax.ShapeDtypeStruct((NPIX, C), jnp.float32),
        scratch_types=dbuf() + dbuf() + [
            pltpu.SemaphoreType.DMA,
            pltpu.SemaphoreType.DMA,
        ],
    )
    return fn(img, xs_flat, ys_flat)


def kernel(X, transformation):
    x_linspace = jnp.linspace(-1.0, 1.0, W)
    y_linspace = jnp.linspace(-1.0, 1.0, H)
    x_coordinates, y_coordinates = jnp.meshgrid(x_linspace, y_linspace)
    x_coordinates = x_coordinates.reshape(-1)
    y_coordinates = y_coordinates.reshape(-1)
    ones = jnp.ones_like(x_coordinates)
    grid = jnp.concatenate([x_coordinates, y_coordinates, ones], axis=0)
    grids = jnp.tile(grid.reshape(-1), (B,)).reshape(B, 3, HW)
    transformations = transformation.reshape(B, 2, 3)
    sampled_grids = jnp.einsum('bij,bjk->bik', transformations, grids)
    x = sampled_grids[:, 0:1, :].reshape(-1).astype(jnp.float32)
    y = sampled_grids[:, 1:2, :].reshape(-1).astype(jnp.float32)
    x = 0.5 * (x + 1.0) * jnp.float32(H)
    y = 0.5 * (y + 1.0) * jnp.float32(W)

    img = X.reshape(NPIX, C)
    out = _sc_interp(img, x, y)
    return out.reshape(B, H, W, C)


# speedup vs baseline: 1.9706x; 1.7104x over previous
import functools

import jax
import jax.numpy as jnp
import numpy as np
from jax import lax
from jax.experimental import pallas as pl
from jax.experimental.pallas import tpu as pltpu
from jax.experimental.pallas import tpu_sc as plsc

B, H, W, C = 4, 224, 224, 192
HW = H * W
NPIX = B * HW
LANES = 16
CH = 64
GROUPS = CH // LANES
NTILES = 32
PTS_PER_TILE = NPIX // NTILES
NCHUNKS = PTS_PER_TILE // CH
CVECS = C // LANES
PROW = 1024


def _corner_slice(corner, cv):
    flat = corner * 256 + cv * LANES
    return flat // 128, flat % 128


def _tec_body(patch, xs_hbm, ys_hbm, out,
              xsv, ysv, idxv, wav, wbv, wcv, wdv, bufp, outb, gsem):
    c_id = lax.axis_index("c")
    s_id = lax.axis_index("s")
    wid = s_id * 2 + c_id
    base0 = wid * PTS_PER_TILE
    batch = wid // (NTILES // B)
    bb = batch * HW

    def chunk_body(t, _):
        start = base0 + t * CH
        pltpu.sync_copy(xs_hbm.at[pl.ds(start, CH)], xsv)
        pltpu.sync_copy(ys_hbm.at[pl.ds(start, CH)], ysv)
        for g in range(GROUPS):
            sl = pl.ds(g * LANES, LANES)
            xs = xsv[sl]
            ys = ysv[sl]
            x0 = xs.astype(jnp.int32)
            y0 = ys.astype(jnp.int32)
            x0c = jnp.clip(x0, 0, W - 1)
            x1c = jnp.clip(x0 + 1, 0, W - 1)
            y0c = jnp.clip(y0, 0, H - 1)
            y1c = jnp.clip(y0 + 1, 0, H - 1)
            x0f = x0c.astype(jnp.float32)
            x1f = x1c.astype(jnp.float32)
            y0f = y0c.astype(jnp.float32)
            y1f = y1c.astype(jnp.float32)
            zero = jnp.zeros((LANES,), jnp.float32)
            eqx = x0c == x1c
            eqy = y0c == y1c
            wxl = jnp.where(eqx, zero, x1f - xs)
            wxr = jnp.where(eqx, zero, xs - x0f)
            wyt = jnp.where(eqy, zero, y1f - ys)
            wyb = jnp.where(eqy, zero, ys - y0f)
            wav[sl] = wxl * wyt
            wbv[sl] = wxl * wyb
            wcv[sl] = wxr * wyt
            wdv[sl] = wxr * wyb
            idxv[sl] = bb + y0c * W + x0c
        pltpu.async_copy(patch.at[idxv], bufp, gsem).wait()

        @plsc.parallel_loop(0, CH, step=1, unroll=4)
        def pt_body(p):
            pidx = jnp.full((LANES,), p, jnp.int32)
            wa = plsc.load_gather(wav, [pidx])
            wb = plsc.load_gather(wbv, [pidx])
            wc = plsc.load_gather(wcv, [pidx])
            wd = plsc.load_gather(wdv, [pidx])
            for cv in range(CVECS):
                ra, ca = _corner_slice(0, cv)
                rc, cc = _corner_slice(1, cv)
                rb, cb = _corner_slice(2, cv)
                rd, cd = _corner_slice(3, cv)
                acc = ((wa * bufp[p, ra, pl.ds(ca, LANES)]
                        + wb * bufp[p, rb, pl.ds(cb, LANES)])
                       + wc * bufp[p, rc, pl.ds(cc, LANES)]) \
                    + wd * bufp[p, rd, pl.ds(cd, LANES)]
                outb[p, pl.ds(cv * LANES, LANES)] = acc

        pltpu.sync_copy(outb, out.at[pl.ds(start, CH)])
        return 0

    lax.fori_loop(0, NCHUNKS, chunk_body, 0)


@jax.jit
def _sc_interp(patch, xs_flat, ys_flat):
    mesh = plsc.VectorSubcoreMesh(core_axis_name="c", subcore_axis_name="s")
    fn = pl.kernel(
        _tec_body,
        mesh=mesh,
        compiler_params=pltpu.CompilerParams(
            needs_layout_passes=False, use_tc_tiling_on_sc=True),
        out_type=jax.ShapeDtypeStruct((NPIX, C), jnp.float32),
        scratch_types=[
            pltpu.VMEM((CH,), jnp.float32),
            pltpu.VMEM((CH,), jnp.float32),
            pltpu.VMEM((CH,), jnp.int32),
            pltpu.VMEM((CH,), jnp.float32),
            pltpu.VMEM((CH,), jnp.float32),
            pltpu.VMEM((CH,), jnp.float32),
            pltpu.VMEM((CH,), jnp.float32),
            pltpu.VMEM((CH, 8, 128), jnp.float32),
            pltpu.VMEM((CH, C), jnp.float32),
            pltpu.SemaphoreType.DMA,
        ],
    )
    return fn(patch, xs_flat, ys_flat)


BLK = 512


def _patch_body(a_ref, b_ref, out_ref):
    a = a_ref[...]
    b = b_ref[...]
    for k, off in enumerate((0, 1, W, W + 1)):
        if off == 0:
            sh = a
        else:
            sh = jnp.concatenate([a[off:], b[:off]], axis=0)
        shp = jnp.pad(sh, ((0, 0), (0, 256 - C)))
        out_ref[:, 2 * k:2 * k + 2, :] = shp.reshape(BLK, 2, 128)


@jax.jit
def _patch_build(imgf):
    nblk = NPIX // BLK
    return pl.pallas_call(
        _patch_body,
        grid=(nblk,),
        in_specs=[
            pl.BlockSpec((BLK, C), lambda i: (i, 0)),
            pl.BlockSpec((BLK, C), lambda i: (jnp.minimum(i + 1, nblk - 1), 0)),
        ],
        out_specs=pl.BlockSpec((BLK, 8, 128), lambda i: (i, 0, 0)),
        out_shape=jax.ShapeDtypeStruct((NPIX, 8, 128), jnp.float32),
    )(imgf, imgf)


def kernel(X, transformation):
    x_linspace = jnp.linspace(-1.0, 1.0, W)
    y_linspace = jnp.linspace(-1.0, 1.0, H)
    x_coordinates, y_coordinates = jnp.meshgrid(x_linspace, y_linspace)
    x_coordinates = x_coordinates.reshape(-1)
    y_coordinates = y_coordinates.reshape(-1)
    ones = jnp.ones_like(x_coordinates)
    grid = jnp.concatenate([x_coordinates, y_coordinates, ones], axis=0)
    grids = jnp.tile(grid.reshape(-1), (B,)).reshape(B, 3, HW)
    transformations = transformation.reshape(B, 2, 3)
    sampled_grids = jnp.einsum('bij,bjk->bik', transformations, grids)
    x = sampled_grids[:, 0:1, :].reshape(-1).astype(jnp.float32)
    y = sampled_grids[:, 1:2, :].reshape(-1).astype(jnp.float32)
    x = 0.5 * (x + 1.0) * jnp.float32(H)
    y = 0.5 * (y + 1.0) * jnp.float32(W)

    imgf = X.reshape(NPIX, C)
    patch = _patch_build(imgf)

    out = _sc_interp(patch, x, y)
    return out.reshape(B, H, W, C)


# speedup vs baseline: 2.1789x; 1.1057x over previous
import functools

import jax
import jax.numpy as jnp
import numpy as np
from jax import lax
from jax.experimental import pallas as pl
from jax.experimental.pallas import tpu as pltpu
from jax.experimental.pallas import tpu_sc as plsc

B, H, W, C = 4, 224, 224, 192
HW = H * W
NPIX = B * HW
LANES = 16
CH = 64
GROUPS = CH // LANES
NTILES = 32
PTS_PER_TILE = NPIX // NTILES
NCHUNKS = PTS_PER_TILE // CH
CVECS = C // LANES
PROW = 1024


def _corner_slice(corner, cv):
    flat = corner * 256 + cv * LANES
    return flat // 128, flat % 128


def _tec_body(patch, xs_hbm, ys_hbm, out,
              xsv, ysv, idxv, wav, wbv, wcv, wdv, bufp, outb, gsem):
    c_id = lax.axis_index("c")
    s_id = lax.axis_index("s")
    wid = s_id * 2 + c_id
    base0 = wid * PTS_PER_TILE
    batch = wid // (NTILES // B)
    bb = batch * HW

    def chunk_body(t, _):
        start = base0 + t * CH
        pltpu.sync_copy(xs_hbm.at[pl.ds(start, CH)], xsv)
        pltpu.sync_copy(ys_hbm.at[pl.ds(start, CH)], ysv)
        for g in range(GROUPS):
            sl = pl.ds(g * LANES, LANES)
            xs = xsv[sl]
            ys = ysv[sl]
            x0 = xs.astype(jnp.int32)
            y0 = ys.astype(jnp.int32)
            x0c = jnp.clip(x0, 0, W - 1)
            x1c = jnp.clip(x0 + 1, 0, W - 1)
            y0c = jnp.clip(y0, 0, H - 1)
            y1c = jnp.clip(y0 + 1, 0, H - 1)
            x0f = x0c.astype(jnp.float32)
            x1f = x1c.astype(jnp.float32)
            y0f = y0c.astype(jnp.float32)
            y1f = y1c.astype(jnp.float32)
            zero = jnp.zeros((LANES,), jnp.float32)
            eqx = x0c == x1c
            eqy = y0c == y1c
            wxl = jnp.where(eqx, zero, x1f - xs)
            wxr = jnp.where(eqx, zero, xs - x0f)
            wyt = jnp.where(eqy, zero, y1f - ys)
            wyb = jnp.where(eqy, zero, ys - y0f)
            wav[sl] = wxl * wyt
            wbv[sl] = wxl * wyb
            wcv[sl] = wxr * wyt
            wdv[sl] = wxr * wyb
            idxv[sl] = bb + y0c * W + x0c
        pltpu.async_copy(patch.at[idxv], bufp, gsem).wait()

        @plsc.parallel_loop(0, CH, step=1, unroll=4)
        def pt_body(p):
            pidx = jnp.full((LANES,), p, jnp.int32)
            wa = plsc.load_gather(wav, [pidx])
            wb = plsc.load_gather(wbv, [pidx])
            wc = plsc.load_gather(wcv, [pidx])
            wd = plsc.load_gather(wdv, [pidx])
            for cv in range(CVECS):
                ra, ca = _corner_slice(0, cv)
                rc, cc = _corner_slice(1, cv)
                rb, cb = _corner_slice(2, cv)
                rd, cd = _corner_slice(3, cv)
                acc = ((wa * bufp[p, ra, pl.ds(ca, LANES)]
                        + wb * bufp[p, rb, pl.ds(cb, LANES)])
                       + wc * bufp[p, rc, pl.ds(cc, LANES)]) \
                    + wd * bufp[p, rd, pl.ds(cd, LANES)]
                outb[p, pl.ds(cv * LANES, LANES)] = acc

        pltpu.sync_copy(outb, out.at[pl.ds(start, CH)])
        return 0

    lax.fori_loop(0, NCHUNKS, chunk_body, 0)


@jax.jit
def _sc_interp(patch, xs_flat, ys_flat):
    mesh = plsc.VectorSubcoreMesh(core_axis_name="c", subcore_axis_name="s")
    fn = pl.kernel(
        _tec_body,
        mesh=mesh,
        compiler_params=pltpu.CompilerParams(
            needs_layout_passes=False, use_tc_tiling_on_sc=True),
        out_type=jax.ShapeDtypeStruct((NPIX, C), jnp.float32),
        scratch_types=[
            pltpu.VMEM((CH,), jnp.float32),
            pltpu.VMEM((CH,), jnp.float32),
            pltpu.VMEM((CH,), jnp.int32),
            pltpu.VMEM((CH,), jnp.float32),
            pltpu.VMEM((CH,), jnp.float32),
            pltpu.VMEM((CH,), jnp.float32),
            pltpu.VMEM((CH,), jnp.float32),
            pltpu.VMEM((CH, 8, 128), jnp.float32),
            pltpu.VMEM((CH, C), jnp.float32),
            pltpu.SemaphoreType.DMA,
        ],
    )
    return fn(patch, xs_flat, ys_flat)


RB = 4
RBW = RB * W


def _patch_body(a_ref, b_ref, out_ref):
    a = a_ref[...].reshape(RBW, C)
    b = b_ref[...].reshape(RBW, C)
    for k, off in enumerate((0, 1, W, W + 1)):
        if off == 0:
            sh = a
        else:
            sh = jnp.concatenate([a[off:], b[:off]], axis=0)
        shp = jnp.pad(sh, ((0, 0), (0, 256 - C)))
        out_ref[:, 2 * k:2 * k + 2, :] = shp.reshape(RBW, 2, 128)


@jax.jit
def _patch_build(X):
    nrb = H // RB
    return pl.pallas_call(
        _patch_body,
        grid=(B, nrb),
        in_specs=[
            pl.BlockSpec((1, RB, W, C), lambda b, r: (b, r, 0, 0)),
            pl.BlockSpec((1, RB, W, C),
                         lambda b, r: (b, jnp.minimum(r + 1, nrb - 1), 0, 0)),
        ],
        out_specs=pl.BlockSpec((RBW, 8, 128), lambda b, r: (b * nrb + r, 0, 0)),
        out_shape=jax.ShapeDtypeStruct((NPIX, 8, 128), jnp.float32),
    )(X, X)


def kernel(X, transformation):
    x_linspace = jnp.linspace(-1.0, 1.0, W)
    y_linspace = jnp.linspace(-1.0, 1.0, H)
    x_coordinates, y_coordinates = jnp.meshgrid(x_linspace, y_linspace)
    x_coordinates = x_coordinates.reshape(-1)
    y_coordinates = y_coordinates.reshape(-1)
    ones = jnp.ones_like(x_coordinates)
    grid = jnp.concatenate([x_coordinates, y_coordinates, ones], axis=0)
    grids = jnp.tile(grid.reshape(-1), (B,)).reshape(B, 3, HW)
    transformations = transformation.reshape(B, 2, 3)
    sampled_grids = jnp.einsum('bij,bjk->bik', transformations, grids)
    x = sampled_grids[:, 0:1, :].reshape(-1).astype(jnp.float32)
    y = sampled_grids[:, 1:2, :].reshape(-1).astype(jnp.float32)
    x = 0.5 * (x + 1.0) * jnp.float32(H)
    y = 0.5 * (y + 1.0) * jnp.float32(W)

    patch = _patch_build(X)

    out = _sc_interp(patch, x, y)
    return out.reshape(B, H, W, C)


# speedup vs baseline: 2.2166x; 1.0173x over previous
import functools

import jax
import jax.numpy as jnp
import numpy as np
from jax import lax
from jax.experimental import pallas as pl
from jax.experimental.pallas import tpu as pltpu
from jax.experimental.pallas import tpu_sc as plsc

B, H, W, C = 4, 224, 224, 192
HW = H * W
NPIX = B * HW
LANES = 16
CH = 64
GROUPS = CH // LANES
NTILES = 32
PTS_PER_TILE = NPIX // NTILES
NCHUNKS = PTS_PER_TILE // CH
CVECS = C // LANES
PROW = 1024


def _corner_slice(corner, cv):
    flat = corner * 256 + cv * LANES
    return flat // 128, flat % 128


def _tec_body(patch, xs_hbm, ys_hbm, out,
              xsv, ysv, idxv, wav, wbv, wcv, wdv, bufp, outb, gsem):
    c_id = lax.axis_index("c")
    s_id = lax.axis_index("s")
    wid = s_id * 2 + c_id
    base0 = wid * PTS_PER_TILE
    batch = wid // (NTILES // B)
    bb = batch * HW

    def chunk_body(t, _):
        start = base0 + t * CH
        pltpu.sync_copy(xs_hbm.at[pl.ds(start, CH)], xsv)
        pltpu.sync_copy(ys_hbm.at[pl.ds(start, CH)], ysv)
        for g in range(GROUPS):
            sl = pl.ds(g * LANES, LANES)
            xs = xsv[sl]
            ys = ysv[sl]
            x0 = xs.astype(jnp.int32)
            y0 = ys.astype(jnp.int32)
            x0c = jnp.clip(x0, 0, W - 1)
            x1c = jnp.clip(x0 + 1, 0, W - 1)
            y0c = jnp.clip(y0, 0, H - 1)
            y1c = jnp.clip(y0 + 1, 0, H - 1)
            x0f = x0c.astype(jnp.float32)
            x1f = x1c.astype(jnp.float32)
            y0f = y0c.astype(jnp.float32)
            y1f = y1c.astype(jnp.float32)
            zero = jnp.zeros((LANES,), jnp.float32)
            eqx = x0c == x1c
            eqy = y0c == y1c
            wxl = jnp.where(eqx, zero, x1f - xs)
            wxr = jnp.where(eqx, zero, xs - x0f)
            wyt = jnp.where(eqy, zero, y1f - ys)
            wyb = jnp.where(eqy, zero, ys - y0f)
            wav[sl] = wxl * wyt
            wbv[sl] = wxl * wyb
            wcv[sl] = wxr * wyt
            wdv[sl] = wxr * wyb
            idxv[sl] = bb + y0c * W + x0c
        pltpu.async_copy(patch.at[idxv], bufp, gsem).wait()

        @plsc.parallel_loop(0, CH, step=1, unroll=4)
        def pt_body(p):
            pidx = jnp.full((LANES,), p, jnp.int32)
            wa = plsc.load_gather(wav, [pidx])
            wb = plsc.load_gather(wbv, [pidx])
            wc = plsc.load_gather(wcv, [pidx])
            wd = plsc.load_gather(wdv, [pidx])
            for cv in range(CVECS):
                ra, ca = _corner_slice(0, cv)
                rc, cc = _corner_slice(1, cv)
                rb, cb = _corner_slice(2, cv)
                rd, cd = _corner_slice(3, cv)
                acc = ((wa * bufp[p, ra, pl.ds(ca, LANES)]
                        + wb * bufp[p, rb, pl.ds(cb, LANES)])
                       + wc * bufp[p, rc, pl.ds(cc, LANES)]) \
                    + wd * bufp[p, rd, pl.ds(cd, LANES)]
                outb[p, pl.ds(cv * LANES, LANES)] = acc

        pltpu.sync_copy(outb, out.at[pl.ds(start, CH)])
        return 0

    lax.fori_loop(0, NCHUNKS, chunk_body, 0)


@jax.jit
def _sc_interp(patch, xs_flat, ys_flat):
    mesh = plsc.VectorSubcoreMesh(core_axis_name="c", subcore_axis_name="s")
    fn = pl.kernel(
        _tec_body,
        mesh=mesh,
        compiler_params=pltpu.CompilerParams(
            needs_layout_passes=False, use_tc_tiling_on_sc=True),
        out_type=jax.ShapeDtypeStruct((NPIX, C), jnp.float32),
        scratch_types=[
            pltpu.VMEM((CH,), jnp.float32),
            pltpu.VMEM((CH,), jnp.float32),
            pltpu.VMEM((CH,), jnp.int32),
            pltpu.VMEM((CH,), jnp.float32),
            pltpu.VMEM((CH,), jnp.float32),
            pltpu.VMEM((CH,), jnp.float32),
            pltpu.VMEM((CH,), jnp.float32),
            pltpu.VMEM((CH, 8, 128), jnp.float32),
            pltpu.VMEM((CH, C), jnp.float32),
            pltpu.SemaphoreType.DMA,
        ],
    )
    return fn(patch, xs_flat, ys_flat)


RB = 8
RBW = RB * W


def _patch_body(a_ref, b_ref, out_ref):
    a = a_ref[...].reshape(RBW, C)
    b = b_ref[...].reshape(RBW, C)
    for k, off in enumerate((0, 1, W, W + 1)):
        if off == 0:
            sh = a
        else:
            sh = jnp.concatenate([a[off:], b[:off]], axis=0)
        shp = jnp.pad(sh, ((0, 0), (0, 256 - C)))
        out_ref[:, 2 * k:2 * k + 2, :] = shp.reshape(RBW, 2, 128)


@jax.jit
def _patch_build(X):
    nrb = H // RB
    return pl.pallas_call(
        _patch_body,
        grid=(B, nrb),
        in_specs=[
            pl.BlockSpec((1, RB, W, C), lambda b, r: (b, r, 0, 0)),
            pl.BlockSpec((1, RB, W, C),
                         lambda b, r: (b, jnp.minimum(r + 1, nrb - 1), 0, 0)),
        ],
        out_specs=pl.BlockSpec((RBW, 8, 128), lambda b, r: (b * nrb + r, 0, 0)),
        out_shape=jax.ShapeDtypeStruct((NPIX, 8, 128), jnp.float32),
    )(X, X)


def kernel(X, transformation):
    x_linspace = jnp.linspace(-1.0, 1.0, W)
    y_linspace = jnp.linspace(-1.0, 1.0, H)
    x_coordinates, y_coordinates = jnp.meshgrid(x_linspace, y_linspace)
    x_coordinates = x_coordinates.reshape(-1)
    y_coordinates = y_coordinates.reshape(-1)
    ones = jnp.ones_like(x_coordinates)
    grid = jnp.concatenate([x_coordinates, y_coordinates, ones], axis=0)
    grids = jnp.tile(grid.reshape(-1), (B,)).reshape(B, 3, HW)
    transformations = transformation.reshape(B, 2, 3)
    sampled_grids = jnp.einsum('bij,bjk->bik', transformations, grids)
    x = sampled_grids[:, 0:1, :].reshape(-1).astype(jnp.float32)
    y = sampled_grids[:, 1:2, :].reshape(-1).astype(jnp.float32)
    x = 0.5 * (x + 1.0) * jnp.float32(H)
    y = 0.5 * (y + 1.0) * jnp.float32(W)

    patch = _patch_build(X)

    out = _sc_interp(patch, x, y)
    return out.reshape(B, H, W, C)


# speedup vs baseline: 2.2280x; 1.0051x over previous
import functools

import jax
import jax.numpy as jnp
import numpy as np
from jax import lax
from jax.experimental import pallas as pl
from jax.experimental.pallas import tpu as pltpu
from jax.experimental.pallas import tpu_sc as plsc

B, H, W, C = 4, 224, 224, 192
HW = H * W
NPIX = B * HW
LANES = 16
CH = 64
GROUPS = CH // LANES
NTILES = 32
PTS_PER_TILE = NPIX // NTILES
NCHUNKS = PTS_PER_TILE // CH
CVECS = C // LANES
PROW = 1024


def _corner_slice(corner, cv):
    flat = corner * 256 + cv * LANES
    return flat // 128, flat % 128


def _tec_body(patch, xs_hbm, ys_hbm, out,
              xsv, ysv, idxv, wav, wbv, wcv, wdv, bufp, outb, gsem):
    c_id = lax.axis_index("c")
    s_id = lax.axis_index("s")
    wid = s_id * 2 + c_id
    base0 = wid * PTS_PER_TILE
    batch = wid // (NTILES // B)
    bb = batch * HW

    def chunk_body(t, _):
        start = base0 + t * CH
        pltpu.sync_copy(xs_hbm.at[pl.ds(start, CH)], xsv)
        pltpu.sync_copy(ys_hbm.at[pl.ds(start, CH)], ysv)
        for g in range(GROUPS):
            sl = pl.ds(g * LANES, LANES)
            xs = xsv[sl]
            ys = ysv[sl]
            x0 = xs.astype(jnp.int32)
            y0 = ys.astype(jnp.int32)
            x0c = jnp.clip(x0, 0, W - 1)
            x1c = jnp.clip(x0 + 1, 0, W - 1)
            y0c = jnp.clip(y0, 0, H - 1)
            y1c = jnp.clip(y0 + 1, 0, H - 1)
            x0f = x0c.astype(jnp.float32)
            x1f = x1c.astype(jnp.float32)
            y0f = y0c.astype(jnp.float32)
            y1f = y1c.astype(jnp.float32)
            zero = jnp.zeros((LANES,), jnp.float32)
            eqx = x0c == x1c
            eqy = y0c == y1c
            wxl = jnp.where(eqx, zero, x1f - xs)
            wxr = jnp.where(eqx, zero, xs - x0f)
            wyt = jnp.where(eqy, zero, y1f - ys)
            wyb = jnp.where(eqy, zero, ys - y0f)
            wav[sl] = wxl * wyt
            wbv[sl] = wxl * wyb
            wcv[sl] = wxr * wyt
            wdv[sl] = wxr * wyb
            idxv[sl] = bb + y0c * W + x0c
        pltpu.async_copy(patch.at[idxv], bufp, gsem).wait()

        @plsc.parallel_loop(0, CH, step=1, unroll=4)
        def pt_body(p):
            pidx = jnp.full((LANES,), p, jnp.int32)
            wa = plsc.load_gather(wav, [pidx])
            wb = plsc.load_gather(wbv, [pidx])
            wc = plsc.load_gather(wcv, [pidx])
            wd = plsc.load_gather(wdv, [pidx])
            for cv in range(CVECS):
                ra, ca = _corner_slice(0, cv)
                rc, cc = _corner_slice(1, cv)
                rb, cb = _corner_slice(2, cv)
                rd, cd = _corner_slice(3, cv)
                acc = ((wa * bufp[p, ra, pl.ds(ca, LANES)]
                        + wb * bufp[p, rb, pl.ds(cb, LANES)])
                       + wc * bufp[p, rc, pl.ds(cc, LANES)]) \
                    + wd * bufp[p, rd, pl.ds(cd, LANES)]
                outb[p, pl.ds(cv * LANES, LANES)] = acc

        pltpu.sync_copy(outb, out.at[pl.ds(start, CH)])
        return 0

    lax.fori_loop(0, NCHUNKS, chunk_body, 0)


@jax.jit
def _sc_interp(patch, xs_flat, ys_flat):
    mesh = plsc.VectorSubcoreMesh(core_axis_name="c", subcore_axis_name="s")
    fn = pl.kernel(
        _tec_body,
        mesh=mesh,
        compiler_params=pltpu.CompilerParams(
            needs_layout_passes=False, use_tc_tiling_on_sc=True),
        out_type=jax.ShapeDtypeStruct((NPIX, C), jnp.float32),
        scratch_types=[
            pltpu.VMEM((CH,), jnp.float32),
            pltpu.VMEM((CH,), jnp.float32),
            pltpu.VMEM((CH,), jnp.int32),
            pltpu.VMEM((CH,), jnp.float32),
            pltpu.VMEM((CH,), jnp.float32),
            pltpu.VMEM((CH,), jnp.float32),
            pltpu.VMEM((CH,), jnp.float32),
            pltpu.VMEM((CH, 8, 128), jnp.float32),
            pltpu.VMEM((CH, C), jnp.float32),
            pltpu.SemaphoreType.DMA,
        ],
    )
    return fn(patch, xs_flat, ys_flat)


RB = 16
RBW = RB * W


def _patch_body(a_ref, b_ref, out_ref):
    a = a_ref[...].reshape(RBW, C)
    b = b_ref[...].reshape(RBW, C)
    for k, off in enumerate((0, 1, W, W + 1)):
        if off == 0:
            sh = a
        else:
            sh = jnp.concatenate([a[off:], b[:off]], axis=0)
        shp = jnp.pad(sh, ((0, 0), (0, 256 - C)))
        out_ref[:, 2 * k:2 * k + 2, :] = shp.reshape(RBW, 2, 128)


@jax.jit
def _patch_build(X):
    nrb = H // RB
    return pl.pallas_call(
        _patch_body,
        grid=(B, nrb),
        in_specs=[
            pl.BlockSpec((1, RB, W, C), lambda b, r: (b, r, 0, 0)),
            pl.BlockSpec((1, RB, W, C),
                         lambda b, r: (b, jnp.minimum(r + 1, nrb - 1), 0, 0)),
        ],
        out_specs=pl.BlockSpec((RBW, 8, 128), lambda b, r: (b * nrb + r, 0, 0)),
        out_shape=jax.ShapeDtypeStruct((NPIX, 8, 128), jnp.float32),
    )(X, X)


def kernel(X, transformation):
    x_linspace = jnp.linspace(-1.0, 1.0, W)
    y_linspace = jnp.linspace(-1.0, 1.0, H)
    x_coordinates, y_coordinates = jnp.meshgrid(x_linspace, y_linspace)
    x_coordinates = x_coordinates.reshape(-1)
    y_coordinates = y_coordinates.reshape(-1)
    ones = jnp.ones_like(x_coordinates)
    grid = jnp.concatenate([x_coordinates, y_coordinates, ones], axis=0)
    grids = jnp.tile(grid.reshape(-1), (B,)).reshape(B, 3, HW)
    transformations = transformation.reshape(B, 2, 3)
    sampled_grids = jnp.einsum('bij,bjk->bik', transformations, grids)
    x = sampled_grids[:, 0:1, :].reshape(-1).astype(jnp.float32)
    y = sampled_grids[:, 1:2, :].reshape(-1).astype(jnp.float32)
    x = 0.5 * (x + 1.0) * jnp.float32(H)
    y = 0.5 * (y + 1.0) * jnp.float32(W)

    patch = _patch_build(X)

    out = _sc_interp(patch, x, y)
    return out.reshape(B, H, W, C)


# speedup vs baseline: 2.2457x; 1.0080x over previous
import functools

import jax
import jax.numpy as jnp
import numpy as np
from jax import lax
from jax.experimental import pallas as pl
from jax.experimental.pallas import tpu as pltpu
from jax.experimental.pallas import tpu_sc as plsc

B, H, W, C = 4, 224, 224, 192
HW = H * W
NPIX = B * HW
LANES = 16
CH = 32
GROUPS = CH // LANES
NTILES = 32
PTS_PER_TILE = NPIX // NTILES
NCHUNKS = PTS_PER_TILE // CH
CVECS = C // LANES
PROW = 1024


def _corner_slice(corner, cv):
    flat = corner * 256 + cv * LANES
    return flat // 128, flat % 128


def _tec_body(patch, xs_hbm, ys_hbm, out,
              xsva, ysva, idxva, wav_a, wbv_a, wcv_a, wdv_a, bufpa, outba,
              xsvb, ysvb, idxvb, wav_b, wbv_b, wcv_b, wdv_b, bufpb, outbb,
              gsema, gsemb):
    c_id = lax.axis_index("c")
    s_id = lax.axis_index("s")
    wid = s_id * 2 + c_id
    base0 = wid * PTS_PER_TILE
    batch = wid // (NTILES // B)
    bb = batch * HW

    def emit_idx(t, xsv, ysv, idxv, wav, wbv, wcv, wdv):
        start = base0 + t * CH
        pltpu.sync_copy(xs_hbm.at[pl.ds(start, CH)], xsv)
        pltpu.sync_copy(ys_hbm.at[pl.ds(start, CH)], ysv)
        for g in range(GROUPS):
            sl = pl.ds(g * LANES, LANES)
            xs = xsv[sl]
            ys = ysv[sl]
            x0 = xs.astype(jnp.int32)
            y0 = ys.astype(jnp.int32)
            x0c = jnp.clip(x0, 0, W - 1)
            x1c = jnp.clip(x0 + 1, 0, W - 1)
            y0c = jnp.clip(y0, 0, H - 1)
            y1c = jnp.clip(y0 + 1, 0, H - 1)
            x0f = x0c.astype(jnp.float32)
            x1f = x1c.astype(jnp.float32)
            y0f = y0c.astype(jnp.float32)
            y1f = y1c.astype(jnp.float32)
            zero = jnp.zeros((LANES,), jnp.float32)
            eqx = x0c == x1c
            eqy = y0c == y1c
            wxl = jnp.where(eqx, zero, x1f - xs)
            wxr = jnp.where(eqx, zero, xs - x0f)
            wyt = jnp.where(eqy, zero, y1f - ys)
            wyb = jnp.where(eqy, zero, ys - y0f)
            wav[sl] = wxl * wyt
            wbv[sl] = wxl * wyb
            wcv[sl] = wxr * wyt
            wdv[sl] = wxr * wyb
            idxv[sl] = bb + y0c * W + x0c

    def combine(t, wav, wbv, wcv, wdv, bufp, outb):
        @plsc.parallel_loop(0, CH, step=1, unroll=4)
        def pt_body(p):
            pidx = jnp.full((LANES,), p, jnp.int32)
            wa = plsc.load_gather(wav, [pidx])
            wb = plsc.load_gather(wbv, [pidx])
            wc = plsc.load_gather(wcv, [pidx])
            wd = plsc.load_gather(wdv, [pidx])
            for cv in range(CVECS):
                ra, ca = _corner_slice(0, cv)
                rc, cc = _corner_slice(1, cv)
                rb, cb = _corner_slice(2, cv)
                rd, cd = _corner_slice(3, cv)
                acc = ((wa * bufp[p, ra, pl.ds(ca, LANES)]
                        + wb * bufp[p, rb, pl.ds(cb, LANES)])
                       + wc * bufp[p, rc, pl.ds(cc, LANES)]) \
                    + wd * bufp[p, rd, pl.ds(cd, LANES)]
                outb[p, pl.ds(cv * LANES, LANES)] = acc

        pltpu.sync_copy(outb, out.at[pl.ds(base0 + t * CH, CH)])

    seta = (xsva, ysva, idxva, wav_a, wbv_a, wcv_a, wdv_a)
    setb = (xsvb, ysvb, idxvb, wav_b, wbv_b, wcv_b, wdv_b)

    emit_idx(0, *seta)
    pltpu.async_copy(patch.at[idxva], bufpa, gsema)

    def pair_body(k, _):
        ta = 2 * k
        tb = ta + 1
        emit_idx(tb, *setb)
        pltpu.async_copy(patch.at[idxvb], bufpb, gsemb)
        pltpu.make_async_copy(patch.at[idxva], bufpa, gsema).wait()
        combine(ta, wav_a, wbv_a, wcv_a, wdv_a, bufpa, outba)

        @pl.when(k < NCHUNKS // 2 - 1)
        def _():
            emit_idx(ta + 2, *seta)
            pltpu.async_copy(patch.at[idxva], bufpa, gsema)

        pltpu.make_async_copy(patch.at[idxvb], bufpb, gsemb).wait()
        combine(tb, wav_b, wbv_b, wcv_b, wdv_b, bufpb, outbb)
        return 0

    lax.fori_loop(0, NCHUNKS // 2, pair_body, 0)


@jax.jit
def _sc_interp(patch, xs_flat, ys_flat):
    mesh = plsc.VectorSubcoreMesh(core_axis_name="c", subcore_axis_name="s")
    fn = pl.kernel(
        _tec_body,
        mesh=mesh,
        compiler_params=pltpu.CompilerParams(
            needs_layout_passes=False, use_tc_tiling_on_sc=True),
        out_type=jax.ShapeDtypeStruct((NPIX, C), jnp.float32),
        scratch_types=(
            [pltpu.VMEM((CH,), jnp.float32)] * 2
            + [pltpu.VMEM((CH,), jnp.int32)]
            + [pltpu.VMEM((CH,), jnp.float32)] * 4
            + [pltpu.VMEM((CH, 8, 128), jnp.float32),
               pltpu.VMEM((CH, C), jnp.float32)]
        ) * 2 + [
            pltpu.SemaphoreType.DMA,
            pltpu.SemaphoreType.DMA,
        ],
    )
    return fn(patch, xs_flat, ys_flat)


RB = 16
RBW = RB * W


def _patch_body(a_ref, b_ref, out_ref):
    a = a_ref[...].reshape(RBW, C)
    b = b_ref[...].reshape(RBW, C)
    for k, off in enumerate((0, 1, W, W + 1)):
        if off == 0:
            sh = a
        else:
            sh = jnp.concatenate([a[off:], b[:off]], axis=0)
        shp = jnp.pad(sh, ((0, 0), (0, 256 - C)))
        out_ref[:, 2 * k:2 * k + 2, :] = shp.reshape(RBW, 2, 128)


@jax.jit
def _patch_build(X):
    nrb = H // RB
    return pl.pallas_call(
        _patch_body,
        grid=(B, nrb),
        in_specs=[
            pl.BlockSpec((1, RB, W, C), lambda b, r: (b, r, 0, 0)),
            pl.BlockSpec((1, RB, W, C),
                         lambda b, r: (b, jnp.minimum(r + 1, nrb - 1), 0, 0)),
        ],
        out_specs=pl.BlockSpec((RBW, 8, 128), lambda b, r: (b * nrb + r, 0, 0)),
        out_shape=jax.ShapeDtypeStruct((NPIX, 8, 128), jnp.float32),
    )(X, X)


def kernel(X, transformation):
    x_linspace = jnp.linspace(-1.0, 1.0, W)
    y_linspace = jnp.linspace(-1.0, 1.0, H)
    x_coordinates, y_coordinates = jnp.meshgrid(x_linspace, y_linspace)
    x_coordinates = x_coordinates.reshape(-1)
    y_coordinates = y_coordinates.reshape(-1)
    ones = jnp.ones_like(x_coordinates)
    grid = jnp.concatenate([x_coordinates, y_coordinates, ones], axis=0)
    grids = jnp.tile(grid.reshape(-1), (B,)).reshape(B, 3, HW)
    transformations = transformation.reshape(B, 2, 3)
    sampled_grids = jnp.einsum('bij,bjk->bik', transformations, grids)
    x = sampled_grids[:, 0:1, :].reshape(-1).astype(jnp.float32)
    y = sampled_grids[:, 1:2, :].reshape(-1).astype(jnp.float32)
    x = 0.5 * (x + 1.0) * jnp.float32(H)
    y = 0.5 * (y + 1.0) * jnp.float32(W)

    patch = _patch_build(X)

    out = _sc_interp(patch, x, y)
    return out.reshape(B, H, W, C)
